# baseline (speedup 1.0000x reference)
import functools

import jax
import jax.numpy as jnp
from jax.experimental import pallas as pl
from jax.experimental.pallas import tpu as pltpu

_D = 256
_H = 8
_DH = _D // _H


def _elu1(x):
    return jnp.where(x > 0, x + 1.0, jnp.exp(jnp.minimum(x, 0.0)))


def _mm(a, b):
    return jnp.dot(a, b, preferred_element_type=jnp.float32)


def _ln(x, g, b, eps=1e-5):
    mu = jnp.mean(x, axis=-1, keepdims=True)
    var = jnp.mean((x - mu) ** 2, axis=-1, keepdims=True)
    return (x - mu) * jax.lax.rsqrt(var + eps) * g + b


_SCHEDULE = (
    ('stats', 0, 0, None, ((0, 0),)),
    ('stats', 1, 0, None, ((1, 0),)),
    ('apply', 0, 0, 0, ()),
    ('apply', 1, 0, 1, ((0, 1),)),
    ('apply', 0, 1, 0, ((1, 1), (2, 2))),
    ('apply', 1, 1, 1, ((0, 2),)),
    ('apply', 0, 2, 2, ()),
    ('apply', 1, 2, 0, ((1, 3),)),
    ('apply', 0, 3, 1, ((2, 3),)),
    ('apply', 1, 3, 2, ()),
)


def _net_kernel(f0_ref, f1_ref, wq_ref, wk_ref, wv_ref, wm_ref, w1_ref,
                w2_ref, g1_ref, b1_ref, g2_ref, b2_ref, mask_ref,
                f0o_ref, f1o_ref, f0s, f1s, comb, kss,
                *, seq_len, tl, nt):
    p = pl.program_id(0)
    n = pl.program_id(1)
    t = pl.program_id(2)
    rows = pl.ds(n * seq_len + t * tl, tl)
    mask = mask_ref[...]

    def stats_update(y, slot, layer):
        kf = _elu1(_mm(y, wk_ref[layer]))
        v = _mm(y, wv_ref[layer]) * (1.0 / seq_len)
        kv = jax.lax.dot_general(kf, v, (((0,), (0,)), ((), ())),
                                 preferred_element_type=jnp.float32) * mask
        ks = jnp.sum(kf, axis=0, keepdims=True)
        crows = pl.ds(slot * (2 * _D) + n * _D, _D)
        krow = pl.ds(slot * 2 + n, 1)

        @pl.when(t == 0)
        def _init():
            comb[crows, :_D] = kv
            kss[krow, :] = ks

        @pl.when(t != 0)
        def _acc():
            comb[crows, :_D] += kv
            kss[krow, :] += ks

        @pl.when(t == nt - 1)
        def _fin():
            r = jax.lax.broadcasted_iota(jnp.int32, (_D, _D), 0)
            c = jax.lax.broadcasted_iota(jnp.int32, (_D, _D), 1)
            diag_ks = jnp.where(r == c, kss[krow, :], 0.0)
            comb[crows, _D:] = _mm(diag_ks, mask)

    def apply_body(x, slot, layer):
        qf = _elu1(_mm(x, wq_ref[layer]))
        ad = _mm(qf, comb[pl.ds(slot * (2 * _D) + n * _D, _D), :])
        msg = ad[:, :_D] * (seq_len / (ad[:, _D:] + 1e-6))
        m = _mm(msg, wm_ref[layer])
        m = _ln(m, g1_ref[layer], b1_ref[layer])
        h = _mm(x, w1_ref[layer, :_D, :]) + _mm(m, w1_ref[layer, _D:, :])
        h = jnp.maximum(h, 0.0)
        m2 = _mm(h, w2_ref[layer])
        m2 = _ln(m2, g2_ref[layer], b2_ref[layer])
        return x + m2

    for pi, (kind, feat, layer, cslot, produce) in enumerate(_SCHEDULE):
        in_ref = f0_ref if feat == 0 else f1_ref
        scr = f0s if feat == 0 else f1s
        first_apply = (pi == 2) if feat == 0 else (pi == 3)
        last_apply = (pi == 8) if feat == 0 else (pi == 9)
        out_ref = f0o_ref if feat == 0 else f1o_ref

        @pl.when(p == pi)
        def _pass(kind=kind, layer=layer, cslot=cslot, produce=produce,
                  in_ref=in_ref, scr=scr, first_apply=first_apply,
                  last_apply=last_apply, out_ref=out_ref):
            if kind == 'stats':
                stats_update(in_ref[0], produce[0][0], produce[0][1])
            else:
                x = in_ref[0] if first_apply else scr[rows, :]
                y = apply_body(x, cslot, layer)
                if last_apply:
                    out_ref[0] = y
                else:
                    scr[rows, :] = y
                for s2, l2 in produce:
                    stats_update(y, s2, l2)


def _active01(p, a, b):
    return jnp.logical_or(p == a, p == b)


def kernel(feat0, feat1, Wq, Wk, Wv, Wm, W1, W2, g1, b1, g2, b2):
    n, seq_len, d = feat0.shape
    tl = min(1024, seq_len)
    nt = seq_len // tl
    ids = jnp.arange(d) // _DH
    mask = (ids[:, None] == ids[None, :]).astype(jnp.float32)
    g1r, b1r = g1[:, None, :], b1[:, None, :]
    g2r, b2r = g2[:, None, :], b2[:, None, :]
    bf = jnp.bfloat16
    Wq, Wk, Wv, Wm, W1, W2 = (w.astype(bf) for w in (Wq, Wk, Wv, Wm, W1, W2))

    def tile_map(pa, pb):
        def m(p, i, t):
            act = _active01(p, pa, pb)
            return (jnp.where(act, i, 0), jnp.where(act, t, 0), 0)
        return pl.BlockSpec((1, tl, d), m)

    def out_map(pw, last_i, last_t):
        def m(p, i, t):
            return (jnp.where(p == pw, i, jnp.where(p > pw, last_i, 0)),
                    jnp.where(p == pw, t, jnp.where(p > pw, last_t, 0)), 0)
        return pl.BlockSpec((1, tl, d), m)

    const = lambda shape: pl.BlockSpec(shape, lambda p, i, t: (0,) * len(shape))

    f0o, f1o = pl.pallas_call(
        functools.partial(_net_kernel, seq_len=seq_len, tl=tl, nt=nt),
        grid=(len(_SCHEDULE), n, nt),
        in_specs=[
            tile_map(0, 2),
            tile_map(1, 3),
            const((4, d, d)),
            const((4, d, d)),
            const((4, d, d)),
            const((4, d, d)),
            const((4, 2 * d, 2 * d)),
            const((4, 2 * d, d)),
            const((4, 1, d)),
            const((4, 1, d)),
            const((4, 1, d)),
            const((4, 1, d)),
            const((d, d)),
        ],
        out_specs=[out_map(8, n - 1, nt - 1), out_map(9, n - 1, nt - 1)],
        out_shape=[jax.ShapeDtypeStruct((n, seq_len, d), jnp.float32),
                   jax.ShapeDtypeStruct((n, seq_len, d), jnp.float32)],
        scratch_shapes=[
            pltpu.VMEM((n * seq_len, d), jnp.float32),
            pltpu.VMEM((n * seq_len, d), jnp.float32),
            pltpu.VMEM((3 * n * d, 2 * d), jnp.float32),
            pltpu.VMEM((8, d), jnp.float32),
        ],
        compiler_params=pltpu.CompilerParams(
            dimension_semantics=("arbitrary", "arbitrary", "arbitrary")),
    )(feat0, feat1, Wq, Wk, Wv, Wm, W1, W2, g1r, b1r, g2r, b2r, mask)
    return jnp.concatenate([f0o, f1o], axis=0)


# speedup vs baseline: 1.0423x; 1.0423x over previous
import functools

import jax
import jax.numpy as jnp
from jax.experimental import pallas as pl
from jax.experimental.pallas import tpu as pltpu

_D = 256
_H = 8
_DH = _D // _H


def _elu1(x):
    return jnp.where(x > 0, x + 1.0, jnp.exp(jnp.minimum(x, 0.0)))


def _mm(a, b):
    return jnp.dot(a, b, preferred_element_type=jnp.float32)


def _ln(x, g, b, eps=1e-5):
    mu = jnp.mean(x, axis=-1, keepdims=True)
    var = jnp.mean((x - mu) ** 2, axis=-1, keepdims=True)
    return (x - mu) * jax.lax.rsqrt(var + eps) * g + b


_SCHEDULE = (
    ('stats', 0, 0, None, ((0, 0),)),
    ('stats', 1, 0, None, ((1, 0),)),
    ('apply', 0, 0, 0, ()),
    ('apply', 1, 0, 1, ((0, 1),)),
    ('apply', 0, 1, 0, ((1, 1), (2, 2))),
    ('apply', 1, 1, 1, ((0, 2),)),
    ('apply', 0, 2, 2, ()),
    ('apply', 1, 2, 0, ((1, 3),)),
    ('apply', 0, 3, 1, ((2, 3),)),
    ('apply', 1, 3, 2, ()),
)


def _net_kernel(f0_ref, f1_ref, wq_ref, wk_ref, wv_ref, wm_ref, w1_ref,
                w2_ref, g1_ref, b1_ref, g2_ref, b2_ref, mask_ref,
                f0o_ref, f1o_ref, f0s, f1s, comb, kss,
                *, seq_len, tl, nt):
    p = pl.program_id(0)
    n = pl.program_id(1)
    t = pl.program_id(2)
    rows = pl.ds(n * seq_len + t * tl, tl)
    mask = mask_ref[...]

    def stats_update(y, slot, layer):
        kf = _elu1(_mm(y, wk_ref[layer]))
        v = _mm(y, wv_ref[layer]) * (1.0 / seq_len)
        kv = jax.lax.dot_general(kf, v, (((0,), (0,)), ((), ())),
                                 preferred_element_type=jnp.float32) * mask
        ks = jnp.sum(kf, axis=0, keepdims=True)
        crows = pl.ds(slot * (2 * _D) + n * _D, _D)
        krow = pl.ds(slot * 2 + n, 1)

        @pl.when(t == 0)
        def _init():
            comb[crows, :_D] = kv
            kss[krow, :] = ks

        @pl.when(t != 0)
        def _acc():
            comb[crows, :_D] += kv
            kss[krow, :] += ks

        @pl.when(t == nt - 1)
        def _fin():
            r = jax.lax.broadcasted_iota(jnp.int32, (_D, _D), 0)
            c = jax.lax.broadcasted_iota(jnp.int32, (_D, _D), 1)
            diag_ks = jnp.where(r == c, kss[krow, :], 0.0)
            comb[crows, _D:] = _mm(diag_ks, mask)

    def apply_body(x, slot, layer):
        qf = _elu1(_mm(x, wq_ref[layer]))
        ad = _mm(qf, comb[pl.ds(slot * (2 * _D) + n * _D, _D), :])
        msg = ad[:, :_D] * (seq_len / (ad[:, _D:] + 1e-6))
        m = _mm(msg, wm_ref[layer])
        m = _ln(m, g1_ref[layer], b1_ref[layer])
        h = _mm(x, w1_ref[layer, :_D, :]) + _mm(m, w1_ref[layer, _D:, :])
        h = jnp.maximum(h, 0.0)
        m2 = _mm(h, w2_ref[layer])
        m2 = _ln(m2, g2_ref[layer], b2_ref[layer])
        return x + m2

    for pi, (kind, feat, layer, cslot, produce) in enumerate(_SCHEDULE):
        in_ref = f0_ref if feat == 0 else f1_ref
        scr = f0s if feat == 0 else f1s
        first_apply = (pi == 2) if feat == 0 else (pi == 3)
        last_apply = (pi == 8) if feat == 0 else (pi == 9)
        out_ref = f0o_ref if feat == 0 else f1o_ref

        @pl.when(p == pi)
        def _pass(kind=kind, layer=layer, cslot=cslot, produce=produce,
                  in_ref=in_ref, scr=scr, first_apply=first_apply,
                  last_apply=last_apply, out_ref=out_ref):
            if kind == 'stats':
                stats_update(in_ref[0], produce[0][0], produce[0][1])
            else:
                x = in_ref[0] if first_apply else scr[rows, :]
                y = apply_body(x, cslot, layer)
                if last_apply:
                    out_ref[0] = y
                else:
                    scr[rows, :] = y
                for s2, l2 in produce:
                    stats_update(y, s2, l2)


def _active01(p, a, b):
    return jnp.logical_or(p == a, p == b)


def kernel(feat0, feat1, Wq, Wk, Wv, Wm, W1, W2, g1, b1, g2, b2):
    n, seq_len, d = feat0.shape
    tl = min(1024, seq_len)
    nt = seq_len // tl
    ids = jnp.arange(d) // _DH
    mask = (ids[:, None] == ids[None, :]).astype(jnp.float32)
    g1r, b1r = g1[:, None, :], b1[:, None, :]
    g2r, b2r = g2[:, None, :], b2[:, None, :]

    def tile_map(pa, pb):
        def m(p, i, t):
            act = _active01(p, pa, pb)
            return (jnp.where(act, i, 0), jnp.where(act, t, 0), 0)
        return pl.BlockSpec((1, tl, d), m)

    def out_map(pw, last_i, last_t):
        def m(p, i, t):
            return (jnp.where(p == pw, i, jnp.where(p > pw, last_i, 0)),
                    jnp.where(p == pw, t, jnp.where(p > pw, last_t, 0)), 0)
        return pl.BlockSpec((1, tl, d), m)

    const = lambda shape: pl.BlockSpec(shape, lambda p, i, t: (0,) * len(shape))

    f0o, f1o = pl.pallas_call(
        functools.partial(_net_kernel, seq_len=seq_len, tl=tl, nt=nt),
        grid=(len(_SCHEDULE), n, nt),
        in_specs=[
            tile_map(0, 2),
            tile_map(1, 3),
            const((4, d, d)),
            const((4, d, d)),
            const((4, d, d)),
            const((4, d, d)),
            const((4, 2 * d, 2 * d)),
            const((4, 2 * d, d)),
            const((4, 1, d)),
            const((4, 1, d)),
            const((4, 1, d)),
            const((4, 1, d)),
            const((d, d)),
        ],
        out_specs=[out_map(8, n - 1, nt - 1), out_map(9, n - 1, nt - 1)],
        out_shape=[jax.ShapeDtypeStruct((n, seq_len, d), jnp.float32),
                   jax.ShapeDtypeStruct((n, seq_len, d), jnp.float32)],
        scratch_shapes=[
            pltpu.VMEM((n * seq_len, d), jnp.float32),
            pltpu.VMEM((n * seq_len, d), jnp.float32),
            pltpu.VMEM((3 * n * d, 2 * d), jnp.float32),
            pltpu.VMEM((8, d), jnp.float32),
        ],
        compiler_params=pltpu.CompilerParams(
            dimension_semantics=("arbitrary", "arbitrary", "arbitrary")),
    )(feat0, feat1, Wq, Wk, Wv, Wm, W1, W2, g1r, b1r, g2r, b2r, mask)
    return jnp.concatenate([f0o, f1o], axis=0)
